# Initial kernel scaffold; baseline (speedup 1.0000x reference)
import jax
import jax.numpy as jnp
from jax.experimental import pallas as pl


def kernel(x, y, centers, num_batches_tracked):
    raise NotImplementedError("write your pallas kernel here")


# speedup vs baseline: 3.8607x; 3.8607x over previous
import jax
import jax.numpy as jnp
from jax.experimental import pallas as pl
from jax.experimental.pallas import tpu as pltpu

_N_CLASSES = 1000
_N_CLASSES_PAD = 1024
_N_EMB = 64
_BATCH = 16384
_BB = 2048
_GRID = _BATCH // _BB


def _tc_body(y_ref, x_ref, centers_ref, nbt_ref, out_ref, acc_ref, cnt_ref):
    i = pl.program_id(0)

    @pl.when(i == 0)
    def _init():
        acc_ref[...] = jnp.zeros_like(acc_ref)
        cnt_ref[...] = jnp.zeros_like(cnt_ref)

    y = y_ref[0]
    classes = jax.lax.broadcasted_iota(jnp.int32, (_N_CLASSES_PAD, _BB), 0)
    oh = (classes == y).astype(jnp.bfloat16)

    x = x_ref[...]
    x_hi = x.astype(jnp.bfloat16)
    x_lo = (x - x_hi.astype(jnp.float32)).astype(jnp.bfloat16)

    part = jnp.dot(oh, x_hi, preferred_element_type=jnp.float32)
    part = part + jnp.dot(oh, x_lo, preferred_element_type=jnp.float32)
    acc_ref[...] += part
    cnt_ref[...] += jnp.sum(oh.astype(jnp.float32), axis=1, keepdims=True)

    @pl.when(i == _GRID - 1)
    def _fin():
        counts = cnt_ref[0:_N_CLASSES, :]
        sums = acc_ref[0:_N_CLASSES, :]
        present = counts > 0.0
        denom = jnp.where(present, counts, 1.0)
        mu = sums / denom
        nbt = nbt_ref[0, 0]
        centers = centers_ref[...]
        cma = (mu + centers * nbt) / (nbt + 1.0)
        out_ref[...] = jnp.where(present, cma, centers)


def _update_centers(x, y, centers, nbt):
    y3 = y.reshape(_GRID, 1, _BB)
    nbt2 = nbt.reshape(1, 1)
    return pl.pallas_call(
        _tc_body,
        grid=(_GRID,),
        in_specs=[
            pl.BlockSpec((1, 1, _BB), lambda i: (i, 0, 0)),
            pl.BlockSpec((_BB, _N_EMB), lambda i: (i, 0)),
            pl.BlockSpec((_N_CLASSES, _N_EMB), lambda i: (0, 0)),
            pl.BlockSpec((1, 1), lambda i: (0, 0)),
        ],
        out_specs=pl.BlockSpec((_N_CLASSES, _N_EMB), lambda i: (0, 0)),
        out_shape=jax.ShapeDtypeStruct((_N_CLASSES, _N_EMB), jnp.float32),
        scratch_shapes=[
            pltpu.VMEM((_N_CLASSES_PAD, _N_EMB), jnp.float32),
            pltpu.VMEM((_N_CLASSES_PAD, 1), jnp.float32),
        ],
    )(y3, x, centers, nbt2)


def kernel(x, y, centers, num_batches_tracked):
    new_centers = _update_centers(x, y, centers, num_batches_tracked)
    return (x, new_centers)
